# baseline (speedup 1.0000x reference)
import jax
import jax.numpy as jnp
from jax import lax
from jax.experimental import pallas as pl
from jax.experimental.pallas import tpu as pltpu, tpu_sc as plsc

BATCH = 16384
DIM = 32
BLK = 128
_INFO = plsc.get_sparse_core_info()
_NC, _NS, _L = _INFO.num_cores, _INFO.num_subcores, _INFO.num_lanes
_NW = _NC * _NS
_BPW = BATCH // _NW
_NRING = 8


def _mf_body(uidx_hbm, iidx_hbm, ut_hbm, it_hbm, out_hbm,
             uidx_v, iidx_v, out_v,
             ub0, ub1, ub2, ub3, ub4, ub5, ub6, ub7,
             ib0, ib1, ib2, ib3, ib4, ib5, ib6, ib7,
             su0, su1, su2, su3, su4, su5, su6, su7,
             si0, si1, si2, si3, si4, si5, si6, si7):
    ubufs = (ub0, ub1, ub2, ub3, ub4, ub5, ub6, ub7)
    ibufs = (ib0, ib1, ib2, ib3, ib4, ib5, ib6, ib7)
    usems = (su0, su1, su2, su3, su4, su5, su6, su7)
    isems = (si0, si1, si2, si3, si4, si5, si6, si7)

    wid = lax.axis_index("s") * _NC + lax.axis_index("c")
    base = wid * _BPW

    pltpu.sync_copy(uidx_hbm.at[pl.ds(base, _BPW)], uidx_v)
    pltpu.sync_copy(iidx_hbm.at[pl.ds(base, _BPW)], iidx_v)

    lanes = lax.iota(jnp.int32, _L)

    def fire(uvec, ivec, k, slot):
        ju = pl.multiple_of((uvec[k] >> 7) << 7, BLK)
        ji = pl.multiple_of((ivec[k] >> 7) << 7, BLK)
        pltpu.async_copy(ut_hbm.at[:, pl.ds(ju, BLK)], ubufs[slot],
                         usems[slot])
        pltpu.async_copy(it_hbm.at[:, pl.ds(ji, BLK)], ibufs[slot],
                         isems[slot])

    def chunk(c, carry):
        col0 = c * _L
        uvec = uidx_v[pl.ds(col0, _L)]
        ivec = iidx_v[pl.ds(col0, _L)]
        for k in range(_NRING):
            fire(uvec, ivec, k, k)
        acc = jnp.zeros((_L,), jnp.float32)
        for k in range(_L):
            slot = k % _NRING
            pltpu.make_async_copy(ut_hbm.at[:, pl.ds(0, BLK)], ubufs[slot],
                                  usems[slot]).wait()
            pltpu.make_async_copy(it_hbm.at[:, pl.ds(0, BLK)], ibufs[slot],
                                  isems[slot]).wait()
            cu = jnp.full((_L,), uvec[k] & (BLK - 1), jnp.int32)
            ci = jnp.full((_L,), ivec[k] & (BLK - 1), jnp.int32)
            ulo = plsc.load_gather(ubufs[slot], [lanes, cu])
            uhi = plsc.load_gather(ubufs[slot], [lanes + _L, cu])
            ilo = plsc.load_gather(ibufs[slot], [lanes, ci])
            ihi = plsc.load_gather(ibufs[slot], [lanes + _L, ci])
            s = jnp.sum(ulo * ilo + uhi * ihi)
            acc = jnp.where(lanes == k, s, acc)
            if k + _NRING < _L:
                fire(uvec, ivec, k + _NRING, slot)
        out_v[pl.ds(col0, _L)] = acc
        return carry

    lax.fori_loop(0, _BPW // _L, chunk, 0)
    pltpu.sync_copy(out_v, out_hbm.at[pl.ds(base, _BPW)])


def kernel(x, user_table, item_table):
    user_idx = x[:, 0].astype(jnp.int32)
    item_idx = x[:, 1].astype(jnp.int32)
    mesh = plsc.VectorSubcoreMesh(core_axis_name="c", subcore_axis_name="s")
    run = pl.kernel(
        _mf_body,
        mesh=mesh,
        compiler_params=pltpu.CompilerParams(needs_layout_passes=False),
        out_type=jax.ShapeDtypeStruct((BATCH,), jnp.float32),
        scratch_types=(
            [pltpu.VMEM((_BPW,), jnp.int32)] * 2
            + [pltpu.VMEM((_BPW,), jnp.float32)]
            + [pltpu.VMEM((DIM, BLK), jnp.float32)] * 16
            + [pltpu.SemaphoreType.DMA] * 16
        ),
    )
    return run(user_idx, item_idx, user_table.T, item_table.T)


# speedup vs baseline: 1.0040x; 1.0040x over previous
import jax
import jax.numpy as jnp
from jax import lax
from jax.experimental import pallas as pl
from jax.experimental.pallas import tpu as pltpu, tpu_sc as plsc

BATCH = 16384
DIM = 32
BLK = 128
_INFO = plsc.get_sparse_core_info()
_NC, _NS, _L = _INFO.num_cores, _INFO.num_subcores, _INFO.num_lanes
_NW = _NC * _NS
_BPW = BATCH // _NW
_NRING = 4


def _mf_body(uidx_hbm, iidx_hbm, ut_hbm, it_hbm, out_hbm,
             uidx_v, iidx_v, out_v,
             ub0, ub1, ub2, ub3, ib0, ib1, ib2, ib3,
             su0, su1, su2, su3, si0, si1, si2, si3):
    ubufs = (ub0, ub1, ub2, ub3)
    ibufs = (ib0, ib1, ib2, ib3)
    usems = (su0, su1, su2, su3)
    isems = (si0, si1, si2, si3)

    wid = lax.axis_index("s") * _NC + lax.axis_index("c")
    base = wid * _BPW

    pltpu.sync_copy(uidx_hbm.at[pl.ds(base, _BPW)], uidx_v)
    pltpu.sync_copy(iidx_hbm.at[pl.ds(base, _BPW)], iidx_v)

    lanes = lax.iota(jnp.int32, _L)

    def fire(uvec, ivec, k, slot):
        ju = pl.multiple_of((uvec[k] >> 7) << 7, BLK)
        ji = pl.multiple_of((ivec[k] >> 7) << 7, BLK)
        pltpu.async_copy(ut_hbm.at[:, pl.ds(ju, BLK)], ubufs[slot],
                         usems[slot])
        pltpu.async_copy(it_hbm.at[:, pl.ds(ji, BLK)], ibufs[slot],
                         isems[slot])

    def chunk(c, carry):
        col0 = c * _L
        uvec = uidx_v[pl.ds(col0, _L)]
        ivec = iidx_v[pl.ds(col0, _L)]
        for k in range(_NRING):
            fire(uvec, ivec, k, k)
        acc = jnp.zeros((_L,), jnp.float32)
        for k in range(_L):
            slot = k % _NRING
            pltpu.make_async_copy(ut_hbm.at[:, pl.ds(0, BLK)], ubufs[slot],
                                  usems[slot]).wait()
            pltpu.make_async_copy(it_hbm.at[:, pl.ds(0, BLK)], ibufs[slot],
                                  isems[slot]).wait()
            cu = jnp.full((_L,), uvec[k] & (BLK - 1), jnp.int32)
            ci = jnp.full((_L,), ivec[k] & (BLK - 1), jnp.int32)
            ulo = plsc.load_gather(ubufs[slot], [lanes, cu])
            uhi = plsc.load_gather(ubufs[slot], [lanes + _L, cu])
            ilo = plsc.load_gather(ibufs[slot], [lanes, ci])
            ihi = plsc.load_gather(ibufs[slot], [lanes + _L, ci])
            s = jnp.sum(ulo * ilo + uhi * ihi)
            acc = jnp.where(lanes == k, s, acc)
            if k + _NRING < _L:
                fire(uvec, ivec, k + _NRING, slot)
        out_v[pl.ds(col0, _L)] = acc
        return carry

    lax.fori_loop(0, _BPW // _L, chunk, 0)
    pltpu.sync_copy(out_v, out_hbm.at[pl.ds(base, _BPW)])


def kernel(x, user_table, item_table):
    user_idx = x[:, 0].astype(jnp.int32)
    item_idx = x[:, 1].astype(jnp.int32)
    mesh = plsc.VectorSubcoreMesh(core_axis_name="c", subcore_axis_name="s")
    run = pl.kernel(
        _mf_body,
        mesh=mesh,
        compiler_params=pltpu.CompilerParams(needs_layout_passes=False),
        out_type=jax.ShapeDtypeStruct((BATCH,), jnp.float32),
        scratch_types=(
            [pltpu.VMEM((_BPW,), jnp.int32)] * 2
            + [pltpu.VMEM((_BPW,), jnp.float32)]
            + [pltpu.VMEM((DIM, BLK), jnp.float32)] * 8
            + [pltpu.SemaphoreType.DMA] * 8
        ),
    )
    return run(user_idx, item_idx, user_table.T, item_table.T)
